# baseline (speedup 1.0000x reference)
import dataclasses
import functools

import jax
import jax.numpy as jnp
from jax import lax
from jax.experimental import pallas as pl
from jax.experimental.pallas import tpu as pltpu
from jax.experimental.pallas import tpu_sc as plsc

NC = 2
NS = 16
NW = NC * NS
L = 16
K = 40
BATCH = 10
NRB = 5


def _sc_scatter(N, D, E, W, T):
    EPT = E // NW
    CH = EPT // K
    NB = CH // BATCH
    A = (N // NS) // 8 * 8
    REM = N - NS * A
    TH0 = ((T + 1) // 2 + 7) // 8 * 8
    TH1 = T - TH0

    mesh = plsc.VectorSubcoreMesh(core_axis_name="core", subcore_axis_name="subcore")

    cp = pltpu.CompilerParams()
    if "needs_layout_passes" in pltpu.CompilerParams.__dataclass_fields__:
        cp = dataclasses.replace(cp, needs_layout_passes=False)
    if "use_tc_tiling_on_sc" in pltpu.CompilerParams.__dataclass_fields__:
        cp = dataclasses.replace(cp, use_tc_tiling_on_sc=False)

    @functools.partial(
        pl.kernel,
        mesh=mesh,
        compiler_params=cp,
        out_type=[
            jax.ShapeDtypeStruct((N, D), jnp.float32),
            jax.ShapeDtypeStruct((N, D), jnp.float32),
            jax.ShapeDtypeStruct((N,), jnp.float32),
        ],
        scratch_types=[
            pltpu.VMEM((3, BATCH, K), jnp.int32),
            pltpu.VMEM((3, BATCH, K), jnp.int32),
            pltpu.VMEM((W,), jnp.float32),
            pltpu.VMEM((K, D), jnp.float32),
            pltpu.VMEM((K, D), jnp.float32),
            pltpu.VMEM((K, D), jnp.float32),
            pltpu.VMEM((K, D), jnp.float32),
            pltpu.VMEM((K, D), jnp.float32),
            pltpu.VMEM((N,), jnp.float32),
            pltpu.VMEM((TH0,), jnp.int32),
            pltpu.VMEM_SHARED((N, D), jnp.float32),
            pltpu.SemaphoreType.DMA,
            pltpu.SemaphoreType.DMA,
            pltpu.SemaphoreType.DMA,
            pltpu.SemaphoreType.DMA,
            pltpu.SemaphoreType.DMA,
            pltpu.SemaphoreType.DMA,
            pltpu.SemaphoreType.DMA,
            pltpu.SemaphoreType.DMA,
            pltpu.SemaphoreType.DMA,
            pltpu.SemaphoreType.DMA,
            pltpu.SemaphoreType.DMA,
            pltpu.SemaphoreType.DMA,
        ],
    )
    def k(x_hbm, w_hbm, ipk_hbm, tgt_hbm,
          p0_hbm, p1_hbm, mask_hbm,
          ib0, ib1, w_v, r0, r1, r2, r3, r4, mask_v, tgt_v,
          acc,
          g0, g1, g2, g3, g4, s0, s1, s2, s3, s4, b0, b1):
        c = lax.axis_index("core")
        s = lax.axis_index("subcore")
        t = c * NS + s
        ibufs = (ib0, ib1)
        bufs = (r0, r1, r2, r3, r4)
        gsems = (g0, g1, g2, g3, g4)
        ssems = (s0, s1, s2, s3, s4)
        bsems = (b0, b1)

        pltpu.sync_copy(w_hbm, w_v)

        @pl.loop(0, K)
        def _(r):
            for j in range(D // L):
                r0[r, pl.ds(j * L, L)] = jnp.zeros((L,), jnp.float32)

        base = pl.multiple_of(s * A, 8)
        rows_sl = pl.ds(base, A)
        rem_base = pl.multiple_of(NS * A + s * 8, 8)
        rem_sl = pl.ds(rem_base, 8)

        for j in range(A // K):
            pltpu.sync_copy(r0, acc.at[pl.ds(base + j * K, K)])
        if A % K:
            pltpu.sync_copy(r0.at[pl.ds(0, A % K)],
                            acc.at[pl.ds(base + (A // K) * K, A % K)])

        @pl.when(s < REM // 8)
        def _():
            pltpu.sync_copy(r0.at[pl.ds(0, 8)], acc.at[rem_sl])

        plsc.subcore_barrier()

        def b_start(bi, pb):
            pltpu.async_copy(ipk_hbm.at[t, bi], ibufs[pb], bsems[pb])

        def b_wait(pb):
            pltpu.make_async_copy(ipk_hbm.at[t, 0], ibufs[pb],
                                  bsems[pb]).wait()

        def g_start(b, pb, dib):
            pltpu.async_copy(x_hbm.at[ibufs[pb].at[0, dib]], bufs[b],
                             gsems[b])

        def g_wait(b):
            pltpu.make_async_copy(x_hbm.at[ibufs[0].at[0, 0]], bufs[b],
                                  gsems[b]).wait()

        def s_start(b, pb, dib):
            pltpu.async_copy(bufs[b], acc.at[ibufs[pb].at[1, dib]],
                             ssems[b], add=True)

        def s_wait(b):
            pltpu.make_async_copy(bufs[b], acc.at[ibufs[0].at[1, 0]],
                                  ssems[b]).wait()

        def scale(b, pb, dib):
            buf = bufs[b]
            ib = ibufs[pb]

            @pl.loop(0, K, step=2)
            def _(e0):
                for dd in range(2):
                    e = e0 + dd
                    wi = plsc.load_gather(
                        ib, [jnp.full((L,), 2, jnp.int32),
                             jnp.full((L,), dib, jnp.int32),
                             jnp.full((L,), e, jnp.int32)])
                    bc = plsc.load_gather(w_v, [wi])
                    for j in range(D // L):
                        sl = pl.ds(j * L, L)
                        buf[e, sl] = buf[e, sl] * bc

        def chunk(kk, bi_next, b, pb, dib, in_main):
            if in_main and dib == 4:
                b_start(bi_next, 1 - pb)
            if in_main and dib == BATCH - 1:
                b_wait(1 - pb)

            if in_main:
                @pl.when(kk >= NRB - 1)
                def _():
                    s_wait((b + 1) % NRB)
            else:
                s_wait((b + 1) % NRB)

            nb = (b + 1) % NRB
            npb = pb if dib < BATCH - 1 else 1 - pb
            ndib = (dib + 1) % BATCH
            if in_main:
                g_start(nb, npb, ndib)
            g_wait(b)
            scale(b, pb, dib)
            s_start(b, pb, dib)

        b_start(0, 0)
        b_wait(0)
        g_start(0, 0, 0)

        @pl.loop(0, (NB - 1) // 2)
        def _(oi):
            for d20 in range(2 * BATCH):
                kk = 2 * BATCH * oi + d20
                chunk(kk, 2 * oi + d20 // BATCH + 1,
                      d20 % NRB, d20 // BATCH, d20 % BATCH, True)

        for kk in range(CH - BATCH, CH):
            b = kk % NRB
            pb = (kk // BATCH) % 2
            dib = kk % BATCH
            s_wait((b + 1) % NRB)
            if kk + 1 < CH:
                g_start((b + 1) % NRB, pb, dib + 1)
            g_wait(b)
            scale(b, pb, dib)
            s_start(b, pb, dib)

        for j in range(CH - (NRB - 1), CH):
            s_wait(j % NRB)

        plsc.subcore_barrier()

        @pl.when(c == 0)
        def _():
            pltpu.sync_copy(acc.at[rows_sl], p0_hbm.at[rows_sl])

            @pl.when(s < REM // 8)
            def _():
                pltpu.sync_copy(acc.at[rem_sl], p0_hbm.at[rem_sl])

        @pl.when(c == 1)
        def _():
            pltpu.sync_copy(acc.at[rows_sl], p1_hbm.at[rows_sl])

            @pl.when(s < REM // 8)
            def _():
                pltpu.sync_copy(acc.at[rem_sl], p1_hbm.at[rem_sl])

        @pl.when(jnp.logical_and(c == 0, s == 0))
        def _():
            @pl.loop(0, N // L)
            def _(r):
                mask_v[pl.ds(r * L, L)] = jnp.ones((L,), jnp.float32)

            zeros16 = jnp.zeros((L,), jnp.float32)
            iota16 = lax.iota(jnp.int32, L)

            for toff, tlen in ((0, TH0), (TH0, TH1)):
                pltpu.sync_copy(tgt_hbm.at[pl.ds(toff, tlen)],
                                tgt_v.at[pl.ds(0, tlen)])
                ngroups = (tlen + L - 1) // L

                @pl.loop(0, ngroups)
                def _(g, tlen=tlen):
                    tgt16 = tgt_v[pl.ds(g * L, L)]
                    lanemask = (g * L + iota16) < tlen
                    plsc.store_scatter(mask_v, [tgt16], zeros16,
                                       mask=lanemask)

            pltpu.sync_copy(mask_v, mask_hbm)

    return k


def _combine_body(x_ref, m_ref, p0_ref, p1_ref, o_ref):
    o_ref[...] = x_ref[...] * m_ref[...] + p0_ref[...] + p1_ref[...]


def kernel(layer_input, weights, u, v, weight_idx, targets):
    N, D = layer_input.shape
    (E,) = u.shape
    (W,) = weights.shape
    (T,) = targets.shape
    EPT = E // NW
    NB = (EPT // K) // BATCH

    u = u.astype(jnp.int32).reshape(NW, NB, BATCH, K)
    v = v.astype(jnp.int32).reshape(NW, NB, BATCH, K)
    weight_idx = weight_idx.astype(jnp.int32).reshape(NW, NB, BATCH, K)
    ipk = jnp.stack([u, v, weight_idx], axis=2)
    targets = targets.astype(jnp.int32)

    p0, p1, mask = _sc_scatter(N, D, E, W, T)(
        layer_input, weights, ipk, targets)

    BN = 2000
    out = pl.pallas_call(
        _combine_body,
        grid=(N // BN,),
        in_specs=[
            pl.BlockSpec((BN, D), lambda i: (i, 0)),
            pl.BlockSpec((BN, 1), lambda i: (i, 0)),
            pl.BlockSpec((BN, D), lambda i: (i, 0)),
            pl.BlockSpec((BN, D), lambda i: (i, 0)),
        ],
        out_specs=pl.BlockSpec((BN, D), lambda i: (i, 0)),
        out_shape=jax.ShapeDtypeStruct((N, D), jnp.float32),
    )(layer_input, mask.reshape(N, 1), p0, p1)
    return out


# speedup vs baseline: 1.1820x; 1.1820x over previous
import dataclasses
import functools

import jax
import jax.numpy as jnp
from jax import lax
from jax.experimental import pallas as pl
from jax.experimental.pallas import tpu as pltpu
from jax.experimental.pallas import tpu_sc as plsc

NC = 2
NS = 16
NW = NC * NS
L = 16
K = 100
BATCH = 4
NRB = 3
NIB = 3
MC = 104


def _sc_scatter(N, D, E, W, T, TP):
    EPT = E // NW
    CH = EPT // K
    NB = CH // BATCH
    A = (N // NS) // 8 * 8
    REM = N - NS * A
    MR = (N + MC - 1) // MC

    mesh = plsc.VectorSubcoreMesh(core_axis_name="core", subcore_axis_name="subcore")

    cp = pltpu.CompilerParams()
    if "needs_layout_passes" in pltpu.CompilerParams.__dataclass_fields__:
        cp = dataclasses.replace(cp, needs_layout_passes=False)
    if "use_tc_tiling_on_sc" in pltpu.CompilerParams.__dataclass_fields__:
        cp = dataclasses.replace(cp, use_tc_tiling_on_sc=False)

    @functools.partial(
        pl.kernel,
        mesh=mesh,
        compiler_params=cp,
        out_type=[
            jax.ShapeDtypeStruct((N, D), jnp.float32),
            jax.ShapeDtypeStruct((N, D), jnp.float32),
            jax.ShapeDtypeStruct((MR, MC), jnp.float32),
        ],
        scratch_types=[
            pltpu.VMEM((3, BATCH, K), jnp.int32),
            pltpu.VMEM((3, BATCH, K), jnp.int32),
            pltpu.VMEM((3, BATCH, K), jnp.int32),
            pltpu.VMEM((W,), jnp.float32),
            pltpu.VMEM((K, D), jnp.float32),
            pltpu.VMEM((K, D), jnp.float32),
            pltpu.VMEM((K, D), jnp.float32),
            pltpu.VMEM_SHARED((N, D), jnp.float32),
            pltpu.SemaphoreType.DMA,
            pltpu.SemaphoreType.DMA,
            pltpu.SemaphoreType.DMA,
            pltpu.SemaphoreType.DMA,
            pltpu.SemaphoreType.DMA,
            pltpu.SemaphoreType.DMA,
            pltpu.SemaphoreType.DMA,
            pltpu.SemaphoreType.DMA,
            pltpu.SemaphoreType.DMA,
        ],
    )
    def k(x_hbm, w_hbm, ipk_hbm, tgt_hbm,
          p0_hbm, p1_hbm, mask_hbm,
          ib0, ib1, ib2, w_v, r0, r1, r2, acc,
          g0, g1, g2, s0, s1, s2, b0, b1, b2):
        c = lax.axis_index("core")
        s = lax.axis_index("subcore")
        t = c * NS + s
        ibufs = (ib0, ib1, ib2)
        bufs = (r0, r1, r2)
        gsems = (g0, g1, g2)
        ssems = (s0, s1, s2)
        bsems = (b0, b1, b2)

        pltpu.sync_copy(w_hbm, w_v)

        @pl.loop(0, K)
        def _(r):
            for j in range(D // L):
                r0[r, pl.ds(j * L, L)] = jnp.zeros((L,), jnp.float32)

        base = pl.multiple_of(s * A, 8)
        rows_sl = pl.ds(base, A)
        rem_base = pl.multiple_of(NS * A + s * 8, 8)
        rem_sl = pl.ds(rem_base, 8)

        for j in range(A // K):
            pltpu.sync_copy(r0, acc.at[pl.ds(base + j * K, K)])
        if A % K:
            pltpu.sync_copy(r0.at[pl.ds(0, A % K)],
                            acc.at[pl.ds(base + (A // K) * K, A % K)])

        @pl.when(s < REM // 8)
        def _():
            pltpu.sync_copy(r0.at[pl.ds(0, 8)], acc.at[rem_sl])

        plsc.subcore_barrier()

        def b_start(bi, pb):
            pltpu.async_copy(ipk_hbm.at[t, bi], ibufs[pb], bsems[pb])

        def b_wait(pb):
            pltpu.make_async_copy(ipk_hbm.at[t, 0], ibufs[pb],
                                  bsems[pb]).wait()

        def g_start(b, pb, dib):
            pltpu.async_copy(x_hbm.at[ibufs[pb].at[0, dib]], bufs[b],
                             gsems[b])

        def g_wait(b):
            pltpu.make_async_copy(x_hbm.at[ibufs[0].at[0, 0]], bufs[b],
                                  gsems[b]).wait()

        def s_start(b, pb, dib):
            pltpu.async_copy(bufs[b], acc.at[ibufs[pb].at[1, dib]],
                             ssems[b], add=True)

        def s_wait(b):
            pltpu.make_async_copy(bufs[b], acc.at[ibufs[0].at[1, 0]],
                                  ssems[b]).wait()

        def scale(b, pb, dib):
            buf = bufs[b]
            ib = ibufs[pb]

            @pl.loop(0, K, step=4)
            def _(e0):
                for dd in range(4):
                    e = e0 + dd
                    wi = plsc.load_gather(
                        ib, [jnp.full((L,), 2, jnp.int32),
                             jnp.full((L,), dib, jnp.int32),
                             jnp.full((L,), e, jnp.int32)])
                    bc = plsc.load_gather(w_v, [wi])
                    for j in range(D // L):
                        sl = pl.ds(j * L, L)
                        buf[e, sl] = buf[e, sl] * bc

        def chunk(kk, bi_next, b, pb, dib, in_main):
            if in_main and dib == 1:
                b_start(bi_next, (pb + 1) % NIB)
            if in_main and dib == BATCH - 1:
                b_wait((pb + 1) % NIB)

            if in_main:
                @pl.when(kk >= NRB - 1)
                def _():
                    s_wait((b + 1) % NRB)
            else:
                s_wait((b + 1) % NRB)

            nb = (b + 1) % NRB
            npb = pb if dib < BATCH - 1 else (pb + 1) % NIB
            ndib = (dib + 1) % BATCH
            if in_main:
                g_start(nb, npb, ndib)
            g_wait(b)
            scale(b, pb, dib)
            s_start(b, pb, dib)

        b_start(0, 0)
        b_wait(0)
        g_start(0, 0, 0)

        TR = 3 * BATCH

        @pl.loop(0, (NB - 1) // 3)
        def _(oi):
            for dt in range(TR):
                kk = TR * oi + dt
                chunk(kk, 3 * oi + dt // BATCH + 1,
                      dt % NRB, dt // BATCH % NIB, dt % BATCH, True)

        for kk in range(CH - BATCH, CH):
            b = kk % NRB
            pb = (kk // BATCH) % NIB
            dib = kk % BATCH
            s_wait((b + 1) % NRB)
            if kk + 1 < CH:
                g_start((b + 1) % NRB, pb, dib + 1)
            g_wait(b)
            scale(b, pb, dib)
            s_start(b, pb, dib)

        for j in range(CH - (NRB - 1), CH):
            s_wait(j % NRB)

        plsc.subcore_barrier()

        @pl.when(c == 0)
        def _():
            pltpu.sync_copy(acc.at[rows_sl], p0_hbm.at[rows_sl])

            @pl.when(s < REM // 8)
            def _():
                pltpu.sync_copy(acc.at[rem_sl], p0_hbm.at[rem_sl])

        @pl.when(c == 1)
        def _():
            pltpu.sync_copy(acc.at[rows_sl], p1_hbm.at[rows_sl])

            @pl.when(s < REM // 8)
            def _():
                pltpu.sync_copy(acc.at[rem_sl], p1_hbm.at[rem_sl])

        @pl.when(jnp.logical_and(c == 0, s == 0))
        def _():
            @pl.loop(0, K)
            def _(r):
                for j in range(D // L):
                    r0[r, pl.ds(j * L, L)] = jnp.ones((L,), jnp.float32)

            pltpu.sync_copy(tgt_hbm, r1.at[pl.ds(0, TP // D)])

            zeros16 = jnp.zeros((L,), jnp.float32)
            iota16 = lax.iota(jnp.int32, L)

            @pl.loop(0, TP // D)
            def _(r):
                for gg in range(D // L):
                    t16f = r1[r, pl.ds(gg * L, L)]
                    t16 = plsc.bitcast(t16f, jnp.int32)
                    rowi = t16 // MC
                    coli = t16 - rowi * MC
                    lanemask = (r * D + gg * L + iota16) < T
                    plsc.store_scatter(r0, [rowi, coli], zeros16,
                                       mask=lanemask)

            pltpu.sync_copy(r0.at[pl.ds(0, MR), pl.ds(0, MC)], mask_hbm)

    return k


def _combine_body(x_ref, m_ref, p0_ref, p1_ref, o_ref):
    o_ref[...] = x_ref[...] * m_ref[...] + p0_ref[...] + p1_ref[...]


def kernel(layer_input, weights, u, v, weight_idx, targets):
    N, D = layer_input.shape
    (E,) = u.shape
    (W,) = weights.shape
    (T,) = targets.shape
    EPT = E // NW
    NB = (EPT // K) // BATCH
    TP = ((T + D - 1) // D) * D
    MR = (N + MC - 1) // MC

    u = u.astype(jnp.int32).reshape(NW, NB, BATCH, K)
    v = v.astype(jnp.int32).reshape(NW, NB, BATCH, K)
    weight_idx = weight_idx.astype(jnp.int32).reshape(NW, NB, BATCH, K)
    ipk = jnp.stack([u, v, weight_idx], axis=2)
    tgt_f = jax.lax.bitcast_convert_type(
        jnp.pad(targets.astype(jnp.int32), (0, TP - T)),
        jnp.float32).reshape(TP // D, D)

    p0, p1, mask = _sc_scatter(N, D, E, W, T, TP)(
        layer_input, weights, ipk, tgt_f)

    mask_col = mask.reshape(MR * MC)[:N].reshape(N, 1)

    BN = 2000
    out = pl.pallas_call(
        _combine_body,
        grid=(N // BN,),
        in_specs=[
            pl.BlockSpec((BN, D), lambda i: (i, 0)),
            pl.BlockSpec((BN, 1), lambda i: (i, 0)),
            pl.BlockSpec((BN, D), lambda i: (i, 0)),
            pl.BlockSpec((BN, D), lambda i: (i, 0)),
        ],
        out_specs=pl.BlockSpec((BN, D), lambda i: (i, 0)),
        out_shape=jax.ShapeDtypeStruct((N, D), jnp.float32),
    )(layer_input, mask_col, p0, p1)
    return out


# speedup vs baseline: 1.4046x; 1.1883x over previous
import dataclasses
import functools

import jax
import jax.numpy as jnp
from jax import lax
from jax.experimental import pallas as pl
from jax.experimental.pallas import tpu as pltpu
from jax.experimental.pallas import tpu_sc as plsc

NC = 2
NS = 16
NW = NC * NS
L = 16
K = 100
BATCH = 4
NRB = 3
NIB = 3
MC = 104


def _sc_scatter(N, D, E, W, T, TP):
    EPT = E // NW
    CH = EPT // K
    NB = CH // BATCH
    A = (N // NS) // 8 * 8
    REM = N - NS * A
    MR = (N + MC - 1) // MC

    mesh = plsc.VectorSubcoreMesh(core_axis_name="core", subcore_axis_name="subcore")

    cp = pltpu.CompilerParams()
    if "needs_layout_passes" in pltpu.CompilerParams.__dataclass_fields__:
        cp = dataclasses.replace(cp, needs_layout_passes=False)
    if "use_tc_tiling_on_sc" in pltpu.CompilerParams.__dataclass_fields__:
        cp = dataclasses.replace(cp, use_tc_tiling_on_sc=False)

    @functools.partial(
        pl.kernel,
        mesh=mesh,
        compiler_params=cp,
        out_type=[
            jax.ShapeDtypeStruct((N, D), jnp.float32),
            jax.ShapeDtypeStruct((N, D), jnp.float32),
            jax.ShapeDtypeStruct((MR, MC), jnp.float32),
        ],
        scratch_types=[
            pltpu.VMEM((3, BATCH, K), jnp.int32),
            pltpu.VMEM((3, BATCH, K), jnp.int32),
            pltpu.VMEM((3, BATCH, K), jnp.int32),
            pltpu.VMEM((W,), jnp.float32),
            pltpu.VMEM((K, D), jnp.float32),
            pltpu.VMEM((K, D), jnp.float32),
            pltpu.VMEM((K, D), jnp.float32),
            pltpu.VMEM_SHARED((N, D), jnp.float32),
            pltpu.SemaphoreType.DMA,
            pltpu.SemaphoreType.DMA,
            pltpu.SemaphoreType.DMA,
            pltpu.SemaphoreType.DMA,
            pltpu.SemaphoreType.DMA,
            pltpu.SemaphoreType.DMA,
            pltpu.SemaphoreType.DMA,
            pltpu.SemaphoreType.DMA,
            pltpu.SemaphoreType.DMA,
        ],
    )
    def k(x_hbm, w_hbm, ipk_hbm, tgt_hbm,
          p0_hbm, p1_hbm, mask_hbm,
          ib0, ib1, ib2, w_v, r0, r1, r2, acc,
          g0, g1, g2, s0, s1, s2, b0, b1, b2):
        c = lax.axis_index("core")
        s = lax.axis_index("subcore")
        t = c * NS + s
        ibufs = (ib0, ib1, ib2)
        bufs = (r0, r1, r2)
        gsems = (g0, g1, g2)
        ssems = (s0, s1, s2)
        bsems = (b0, b1, b2)

        pltpu.sync_copy(w_hbm, w_v)

        @pl.loop(0, K)
        def _(r):
            for j in range(D // L):
                r0[r, pl.ds(j * L, L)] = jnp.zeros((L,), jnp.float32)

        base = pl.multiple_of(s * A, 8)
        rows_sl = pl.ds(base, A)
        rem_base = pl.multiple_of(NS * A + s * 8, 8)
        rem_sl = pl.ds(rem_base, 8)

        for j in range(A // K):
            pltpu.sync_copy(r0, acc.at[pl.ds(base + j * K, K)])
        if A % K:
            pltpu.sync_copy(r0.at[pl.ds(0, A % K)],
                            acc.at[pl.ds(base + (A // K) * K, A % K)])

        @pl.when(s < REM // 8)
        def _():
            pltpu.sync_copy(r0.at[pl.ds(0, 8)], acc.at[rem_sl])

        plsc.subcore_barrier()

        def b_start(bi, pb):
            pltpu.async_copy(ipk_hbm.at[t, bi], ibufs[pb], bsems[pb])

        def b_wait(pb):
            pltpu.make_async_copy(ipk_hbm.at[t, 0], ibufs[pb],
                                  bsems[pb]).wait()

        def g_start(b, pb, dib):
            pltpu.async_copy(x_hbm.at[ibufs[pb].at[0, dib]], bufs[b],
                             gsems[b])

        def g_wait(b):
            pltpu.make_async_copy(x_hbm.at[ibufs[0].at[0, 0]], bufs[b],
                                  gsems[b]).wait()

        def s_start(b, pb, dib):
            pltpu.async_copy(bufs[b], acc.at[ibufs[pb].at[1, dib]],
                             ssems[b], add=True)

        def s_wait(b):
            pltpu.make_async_copy(bufs[b], acc.at[ibufs[0].at[1, 0]],
                                  ssems[b]).wait()

        def scale(b, pb, dib):
            buf = bufs[b]
            ib = ibufs[pb]
            iota16 = lax.iota(jnp.int32, L)
            f2 = jnp.full((L,), 2, jnp.int32)
            fdib = jnp.full((L,), dib, jnp.int32)

            def bcast(wv, lane):
                return wv.at[jnp.full((L,), lane, jnp.int32)].get(
                    mode="promise_in_bounds")

            def edge(wv, base, lane):
                bc = bcast(wv, lane)
                for j in range(D // L):
                    sl = pl.ds(j * L, L)
                    buf[base, sl] = buf[base, sl] * bc

            KG = K // L * L

            @pl.loop(0, KG, step=L)
            def _(e0):
                wi = plsc.load_gather(ib, [f2, fdib, e0 + iota16])
                wv = plsc.load_gather(w_v, [wi])

                @pl.loop(0, 4)
                def _(q):
                    for dd in range(4):
                        edge(wv, e0 + q * 4 + dd, q * 4 + dd)

            if K > KG:
                tmask = iota16 < (K - KG)
                wi = plsc.load_gather(ib, [f2, fdib, KG + iota16],
                                      mask=tmask)
                wv = plsc.load_gather(w_v, [wi], mask=tmask)
                for dd in range(K - KG):
                    edge(wv, KG + dd, dd)

        def chunk(kk, bi_next, b, pb, dib, in_main):
            if in_main and dib == 1:
                b_start(bi_next, (pb + 1) % NIB)
            if in_main and dib == BATCH - 1:
                b_wait((pb + 1) % NIB)

            if in_main:
                @pl.when(kk >= NRB - 1)
                def _():
                    s_wait((b + 1) % NRB)
            else:
                s_wait((b + 1) % NRB)

            nb = (b + 1) % NRB
            npb = pb if dib < BATCH - 1 else (pb + 1) % NIB
            ndib = (dib + 1) % BATCH
            if in_main:
                g_start(nb, npb, ndib)
            g_wait(b)
            scale(b, pb, dib)
            s_start(b, pb, dib)

        b_start(0, 0)
        b_wait(0)
        g_start(0, 0, 0)

        TR = 3 * BATCH

        @pl.loop(0, (NB - 1) // 3)
        def _(oi):
            for dt in range(TR):
                kk = TR * oi + dt
                chunk(kk, 3 * oi + dt // BATCH + 1,
                      dt % NRB, dt // BATCH % NIB, dt % BATCH, True)

        for kk in range(CH - BATCH, CH):
            b = kk % NRB
            pb = (kk // BATCH) % NIB
            dib = kk % BATCH
            s_wait((b + 1) % NRB)
            if kk + 1 < CH:
                g_start((b + 1) % NRB, pb, dib + 1)
            g_wait(b)
            scale(b, pb, dib)
            s_start(b, pb, dib)

        for j in range(CH - (NRB - 1), CH):
            s_wait(j % NRB)

        plsc.subcore_barrier()

        @pl.when(c == 0)
        def _():
            pltpu.sync_copy(acc.at[rows_sl], p0_hbm.at[rows_sl])

            @pl.when(s < REM // 8)
            def _():
                pltpu.sync_copy(acc.at[rem_sl], p0_hbm.at[rem_sl])

        @pl.when(c == 1)
        def _():
            pltpu.sync_copy(acc.at[rows_sl], p1_hbm.at[rows_sl])

            @pl.when(s < REM // 8)
            def _():
                pltpu.sync_copy(acc.at[rem_sl], p1_hbm.at[rem_sl])

        @pl.when(jnp.logical_and(c == 0, s == 0))
        def _():
            @pl.loop(0, K)
            def _(r):
                for j in range(D // L):
                    r0[r, pl.ds(j * L, L)] = jnp.ones((L,), jnp.float32)

            pltpu.sync_copy(tgt_hbm, r1.at[pl.ds(0, TP // D)])

            zeros16 = jnp.zeros((L,), jnp.float32)
            iota16 = lax.iota(jnp.int32, L)

            @pl.loop(0, TP // D)
            def _(r):
                for gg in range(D // L):
                    t16f = r1[r, pl.ds(gg * L, L)]
                    t16 = plsc.bitcast(t16f, jnp.int32)
                    rowi = t16 // MC
                    coli = t16 - rowi * MC
                    lanemask = (r * D + gg * L + iota16) < T
                    plsc.store_scatter(r0, [rowi, coli], zeros16,
                                       mask=lanemask)

            pltpu.sync_copy(r0.at[pl.ds(0, MR), pl.ds(0, MC)], mask_hbm)

    return k


def _combine_body(x_ref, m_ref, p0_ref, p1_ref, o_ref):
    o_ref[...] = x_ref[...] * m_ref[...] + p0_ref[...] + p1_ref[...]


def kernel(layer_input, weights, u, v, weight_idx, targets):
    N, D = layer_input.shape
    (E,) = u.shape
    (W,) = weights.shape
    (T,) = targets.shape
    EPT = E // NW
    NB = (EPT // K) // BATCH
    TP = ((T + D - 1) // D) * D
    MR = (N + MC - 1) // MC

    u = u.astype(jnp.int32).reshape(NW, NB, BATCH, K)
    v = v.astype(jnp.int32).reshape(NW, NB, BATCH, K)
    weight_idx = weight_idx.astype(jnp.int32).reshape(NW, NB, BATCH, K)
    ipk = jnp.stack([u, v, weight_idx], axis=2)
    tgt_f = jax.lax.bitcast_convert_type(
        jnp.pad(targets.astype(jnp.int32), (0, TP - T)),
        jnp.float32).reshape(TP // D, D)

    p0, p1, mask = _sc_scatter(N, D, E, W, T, TP)(
        layer_input, weights, ipk, tgt_f)

    mask_col = mask.reshape(MR * MC)[:N].reshape(N, 1)

    BN = 2000
    out = pl.pallas_call(
        _combine_body,
        grid=(N // BN,),
        in_specs=[
            pl.BlockSpec((BN, D), lambda i: (i, 0)),
            pl.BlockSpec((BN, 1), lambda i: (i, 0)),
            pl.BlockSpec((BN, D), lambda i: (i, 0)),
            pl.BlockSpec((BN, D), lambda i: (i, 0)),
        ],
        out_specs=pl.BlockSpec((BN, D), lambda i: (i, 0)),
        out_shape=jax.ShapeDtypeStruct((N, D), jnp.float32),
    )(layer_input, mask_col, p0, p1)
    return out


# speedup vs baseline: 1.5507x; 1.1041x over previous
import dataclasses
import functools

import jax
import jax.numpy as jnp
from jax import lax
from jax.experimental import pallas as pl
from jax.experimental.pallas import tpu as pltpu
from jax.experimental.pallas import tpu_sc as plsc

NC = 2
NS = 16
NW = NC * NS
L = 16
K = 100
BATCH = 4
NRB = 3
NIB = 3
MC = 104


def _sc_scatter(N, D, E, W, T):
    EPT = E // NW
    CH = EPT // K
    NB = CH // BATCH
    A = (N // NS) // 8 * 8
    REM = N - NS * A

    mesh = plsc.VectorSubcoreMesh(core_axis_name="core", subcore_axis_name="subcore")

    cp = pltpu.CompilerParams()
    if "needs_layout_passes" in pltpu.CompilerParams.__dataclass_fields__:
        cp = dataclasses.replace(cp, needs_layout_passes=False)
    if "use_tc_tiling_on_sc" in pltpu.CompilerParams.__dataclass_fields__:
        cp = dataclasses.replace(cp, use_tc_tiling_on_sc=False)

    @functools.partial(
        pl.kernel,
        mesh=mesh,
        compiler_params=cp,
        out_type=[
            jax.ShapeDtypeStruct((N, D), jnp.float32),
            jax.ShapeDtypeStruct((N, D), jnp.float32),
        ],
        scratch_types=[
            pltpu.VMEM((3, BATCH, K), jnp.int32),
            pltpu.VMEM((3, BATCH, K), jnp.int32),
            pltpu.VMEM((3, BATCH, K), jnp.int32),
            pltpu.VMEM((W,), jnp.float32),
            pltpu.VMEM((K, D), jnp.float32),
            pltpu.VMEM((K, D), jnp.float32),
            pltpu.VMEM((K, D), jnp.float32),
            pltpu.VMEM_SHARED((N, D), jnp.float32),
            pltpu.SemaphoreType.DMA,
            pltpu.SemaphoreType.DMA,
            pltpu.SemaphoreType.DMA,
            pltpu.SemaphoreType.DMA,
            pltpu.SemaphoreType.DMA,
            pltpu.SemaphoreType.DMA,
            pltpu.SemaphoreType.DMA,
            pltpu.SemaphoreType.DMA,
            pltpu.SemaphoreType.DMA,
        ],
    )
    def k(x_hbm, w_hbm, ipk_hbm, tgt_hbm,
          p0_hbm, p1_hbm,
          ib0, ib1, ib2, w_v, r0, r1, r2, acc,
          g0, g1, g2, s0, s1, s2, b0, b1, b2):
        c = lax.axis_index("core")
        s = lax.axis_index("subcore")
        t = c * NS + s
        ibufs = (ib0, ib1, ib2)
        bufs = (r0, r1, r2)
        gsems = (g0, g1, g2)
        ssems = (s0, s1, s2)
        bsems = (b0, b1, b2)

        pltpu.sync_copy(w_hbm, w_v)

        @pl.loop(0, K)
        def _(r):
            for j in range(D // L):
                r0[r, pl.ds(j * L, L)] = jnp.zeros((L,), jnp.float32)

        base = pl.multiple_of(s * A, 8)
        rows_sl = pl.ds(base, A)
        rem_base = pl.multiple_of(NS * A + s * 8, 8)
        rem_sl = pl.ds(rem_base, 8)

        @pl.when(c == 0)
        def _():
            pltpu.sync_copy(x_hbm.at[rows_sl], acc.at[rows_sl])

            @pl.when(s < REM // 8)
            def _():
                pltpu.sync_copy(x_hbm.at[rem_sl], acc.at[rem_sl])

        @pl.when(c == 1)
        def _():
            for j in range(A // K):
                pltpu.sync_copy(r0, acc.at[pl.ds(base + j * K, K)])
            if A % K:
                pltpu.sync_copy(r0.at[pl.ds(0, A % K)],
                                acc.at[pl.ds(base + (A // K) * K, A % K)])

            @pl.when(s < REM // 8)
            def _():
                pltpu.sync_copy(r0.at[pl.ds(0, 8)], acc.at[rem_sl])

        plsc.subcore_barrier()

        @pl.when(c == 0)
        def _():
            pltpu.sync_copy(tgt_hbm.at[s], ib0.at[0])
            for j in range(BATCH):
                pltpu.sync_copy(r0, acc.at[ib0.at[0, j]])

        plsc.subcore_barrier()

        def b_start(bi, pb):
            pltpu.async_copy(ipk_hbm.at[t, bi], ibufs[pb], bsems[pb])

        def b_wait(pb):
            pltpu.make_async_copy(ipk_hbm.at[t, 0], ibufs[pb],
                                  bsems[pb]).wait()

        def g_start(b, pb, dib):
            pltpu.async_copy(x_hbm.at[ibufs[pb].at[0, dib]], bufs[b],
                             gsems[b])

        def g_wait(b):
            pltpu.make_async_copy(x_hbm.at[ibufs[0].at[0, 0]], bufs[b],
                                  gsems[b]).wait()

        def s_start(b, pb, dib):
            pltpu.async_copy(bufs[b], acc.at[ibufs[pb].at[1, dib]],
                             ssems[b], add=True)

        def s_wait(b):
            pltpu.make_async_copy(bufs[b], acc.at[ibufs[0].at[1, 0]],
                                  ssems[b]).wait()

        def scale(b, pb, dib):
            buf = bufs[b]
            ib = ibufs[pb]
            iota16 = lax.iota(jnp.int32, L)
            f2 = jnp.full((L,), 2, jnp.int32)
            fdib = jnp.full((L,), dib, jnp.int32)

            def bcast(wv, lane):
                return wv.at[jnp.full((L,), lane, jnp.int32)].get(
                    mode="promise_in_bounds")

            def edge(wv, base, lane):
                bc = bcast(wv, lane)
                for j in range(D // L):
                    sl = pl.ds(j * L, L)
                    buf[base, sl] = buf[base, sl] * bc

            KG = K // L * L

            @pl.loop(0, KG, step=L)
            def _(e0):
                wi = plsc.load_gather(ib, [f2, fdib, e0 + iota16])
                wv = plsc.load_gather(w_v, [wi])

                @pl.loop(0, 4)
                def _(q):
                    for dd in range(4):
                        edge(wv, e0 + q * 4 + dd, q * 4 + dd)

            if K > KG:
                tmask = iota16 < (K - KG)
                wi = plsc.load_gather(ib, [f2, fdib, KG + iota16],
                                      mask=tmask)
                wv = plsc.load_gather(w_v, [wi], mask=tmask)
                for dd in range(K - KG):
                    edge(wv, KG + dd, dd)

        def chunk(kk, bi_next, b, pb, dib, in_main):
            if in_main and dib == 1:
                b_start(bi_next, (pb + 1) % NIB)
            if in_main and dib == BATCH - 1:
                b_wait((pb + 1) % NIB)

            if in_main:
                @pl.when(kk >= NRB - 1)
                def _():
                    s_wait((b + 1) % NRB)
            else:
                s_wait((b + 1) % NRB)

            nb = (b + 1) % NRB
            npb = pb if dib < BATCH - 1 else (pb + 1) % NIB
            ndib = (dib + 1) % BATCH
            if in_main:
                g_start(nb, npb, ndib)
            g_wait(b)
            scale(b, pb, dib)
            s_start(b, pb, dib)

        b_start(0, 0)
        b_wait(0)
        g_start(0, 0, 0)

        TR = 3 * BATCH

        @pl.loop(0, (NB - 1) // 3)
        def _(oi):
            for dt in range(TR):
                kk = TR * oi + dt
                chunk(kk, 3 * oi + dt // BATCH + 1,
                      dt % NRB, dt // BATCH % NIB, dt % BATCH, True)

        for kk in range(CH - BATCH, CH):
            b = kk % NRB
            pb = (kk // BATCH) % NIB
            dib = kk % BATCH
            s_wait((b + 1) % NRB)
            if kk + 1 < CH:
                g_start((b + 1) % NRB, pb, dib + 1)
            g_wait(b)
            scale(b, pb, dib)
            s_start(b, pb, dib)

        for j in range(CH - (NRB - 1), CH):
            s_wait(j % NRB)

        plsc.subcore_barrier()

        @pl.when(c == 0)
        def _():
            pltpu.sync_copy(acc.at[rows_sl], p0_hbm.at[rows_sl])

            @pl.when(s < REM // 8)
            def _():
                pltpu.sync_copy(acc.at[rem_sl], p0_hbm.at[rem_sl])

        @pl.when(c == 1)
        def _():
            pltpu.sync_copy(acc.at[rows_sl], p1_hbm.at[rows_sl])

            @pl.when(s < REM // 8)
            def _():
                pltpu.sync_copy(acc.at[rem_sl], p1_hbm.at[rem_sl])

    return k


def _combine_body(p0_ref, p1_ref, o_ref):
    o_ref[...] = p0_ref[...] + p1_ref[...]


def kernel(layer_input, weights, u, v, weight_idx, targets):
    N, D = layer_input.shape
    (E,) = u.shape
    (W,) = weights.shape
    (T,) = targets.shape
    EPT = E // NW
    NB = (EPT // K) // BATCH
    TPAD = NS * BATCH * K

    u = u.astype(jnp.int32).reshape(NW, NB, BATCH, K)
    v = v.astype(jnp.int32).reshape(NW, NB, BATCH, K)
    weight_idx = weight_idx.astype(jnp.int32).reshape(NW, NB, BATCH, K)
    ipk = jnp.stack([u, v, weight_idx], axis=2)
    tgt_i = targets.astype(jnp.int32)
    tgt_pad = jnp.concatenate(
        [tgt_i, jnp.broadcast_to(tgt_i[:1], (TPAD - T,))]
    ).reshape(NS, BATCH, K)

    p0, p1 = _sc_scatter(N, D, E, W, T)(
        layer_input, weights, ipk, tgt_pad)

    BN = 2000
    out = pl.pallas_call(
        _combine_body,
        grid=(N // BN,),
        in_specs=[
            pl.BlockSpec((BN, D), lambda i: (i, 0)),
            pl.BlockSpec((BN, D), lambda i: (i, 0)),
        ],
        out_specs=pl.BlockSpec((BN, D), lambda i: (i, 0)),
        out_shape=jax.ShapeDtypeStruct((N, D), jnp.float32),
    )(p0, p1)
    return out
